# baseline (device time: 123735 ns/iter reference)
import jax
import jax.numpy as jnp
from jax import lax
from jax.experimental import pallas as pl
from jax.experimental.pallas import tpu as pltpu

N_DEV = 16
HEADS_PER = 4
DH = 64
SQ_L = 128
SKV = 128
B = 2
E = 512
HL = HEADS_PER * DH


def kernel(x, Wq, K_ext, V_ext, Wo):
    xb = x.astype(jnp.bfloat16)
    wq = Wq.astype(jnp.bfloat16)
    wot = Wo.T.astype(jnp.bfloat16)
    eye4 = jnp.eye(HEADS_PER, dtype=jnp.float32)
    k5 = K_ext.reshape(B, SKV, N_DEV, HEADS_PER, DH)
    v5 = V_ext.reshape(B, SKV, N_DEV, HEADS_PER, DH)
    bigkT = jnp.einsum('bkjhd,hg->jbhdgk', k5, eye4).reshape(
        N_DEV, B, HL, HEADS_PER * SKV).astype(jnp.bfloat16)
    bigv = jnp.einsum('bkjhd,hg->jbhkgd', v5, eye4).reshape(
        N_DEV, B, HEADS_PER * SKV, HL).astype(jnp.bfloat16)

    def body(x_ref, wq_ref, wot_ref, bigkT_ref, bigv_ref, out_ref,
             buf, acc, qs, cs, send_sems, recv_sems, lsend_sems, lrecv_sems):
        my = lax.axis_index("i")
        left = lax.rem(my - 1 + N_DEV, N_DEV)
        right = lax.rem(my + 1, N_DEV)

        bsem = pltpu.get_barrier_semaphore()
        for nbr in (left, right):
            pl.semaphore_signal(bsem, inc=1, device_id=(nbr,),
                                device_id_type=pl.DeviceIdType.MESH)
        pl.semaphore_wait(bsem, 2)

        buf[my, 0] = wq_ref[...]
        buf[my, 1] = wot_ref[...]
        acc[...] = jnp.zeros_like(acc)

        x2d = x_ref[...].reshape(B * SQ_L, E)

        HK = HEADS_PER * SKV
        q_ids = my * SQ_L + lax.broadcasted_iota(jnp.int32, (SQ_L, HK), 0)
        k_ids = lax.rem(lax.broadcasted_iota(jnp.int32, (SQ_L, HK), 1), SKV)
        qb = q_ids // 64
        kb = k_ids // 64
        mask = (qb == kb) | (kb == 0) | (lax.rem(qb + kb, 3) == 0)
        neg_big = jnp.where(mask, 0.0, -1e9).astype(jnp.float32)

        r0 = lax.broadcasted_iota(jnp.int32, (HK, HEADS_PER), 0) // SKV
        c0 = lax.broadcasted_iota(jnp.int32, (HK, HEADS_PER), 1)
        ones_blk = jnp.where(r0 == c0, 1.0, 0.0).astype(jnp.bfloat16)
        r1 = lax.broadcasted_iota(jnp.int32, (HEADS_PER, HK), 0)
        c1 = lax.broadcasted_iota(jnp.int32, (HEADS_PER, HK), 1) // SKV
        ind_blk = jnp.where(r1 == c1, 1.0, 0.0).astype(jnp.bfloat16)

        mm = lambda a, b_, dims: lax.dot_general(
            a, b_, (dims, ((), ())), preferred_element_type=jnp.float32)

        def compute_block(j):
            wq_j = buf[j, 0]
            qs[...] = mm(x2d, wq_j, ((1,), (0,))).astype(jnp.bfloat16)
            for b in range(B):
                q_b = qs[b * SQ_L:(b + 1) * SQ_L, :]
                s = mm(q_b, bigkT_ref[j, b], ((1,), (0,)))
                e = jnp.exp(s * 0.125 + neg_big)
                eb = e.astype(jnp.bfloat16)
                denom = mm(eb, ones_blk, ((1,), (0,)))
                dinv = (1.0 / denom).astype(jnp.bfloat16)
                dbig = mm(dinv, ind_blk, ((1,), (0,)))
                w = eb * dbig.astype(jnp.bfloat16)
                c = mm(w, bigv_ref[j, b], ((1,), (0,)))
                cs[b * SQ_L:(b + 1) * SQ_L, :] = c.astype(jnp.bfloat16)
            acc[...] = acc[...] + mm(cs[...], buf[j, 1], ((1,), (1,)))

        R_HOPS = N_DEV // 2
        L_HOPS = N_DEV - 1 - R_HOPS

        def hop(direction_dev, slot, s_sems, r_sems, h):
            rdma = pltpu.make_async_remote_copy(
                src_ref=buf.at[slot],
                dst_ref=buf.at[slot],
                send_sem=s_sems.at[h],
                recv_sem=r_sems.at[h],
                device_id=(direction_dev,),
                device_id_type=pl.DeviceIdType.MESH,
            )
            rdma.start()
            return rdma

        for h in range(R_HOPS):
            r_rdma = hop(right, lax.rem(my - h + N_DEV, N_DEV),
                         send_sems, recv_sems, h)
            l_rdma = None
            if h < L_HOPS:
                l_rdma = hop(left, lax.rem(my + h, N_DEV),
                             lsend_sems, lrecv_sems, h)
            if h == 0:
                compute_block(my)
            else:
                compute_block(lax.rem(my - h + N_DEV, N_DEV))
                compute_block(lax.rem(my + min(h, L_HOPS), N_DEV))
            r_rdma.wait_send()
            r_rdma.wait_recv()
            if l_rdma is not None:
                l_rdma.wait_send()
                l_rdma.wait_recv()
        compute_block(lax.rem(my + R_HOPS, N_DEV))

        out_ref[...] = acc[...].reshape(B, SQ_L, E)

    return pl.pallas_call(
        body,
        out_shape=jax.ShapeDtypeStruct((B, SQ_L, E), jnp.float32),
        in_specs=[pl.BlockSpec(memory_space=pltpu.VMEM)] * 5,
        out_specs=pl.BlockSpec(memory_space=pltpu.VMEM),
        scratch_shapes=[
            pltpu.VMEM((N_DEV, 2, E, HL), jnp.bfloat16),
            pltpu.VMEM((B * SQ_L, E), jnp.float32),
            pltpu.VMEM((B * SQ_L, HL), jnp.bfloat16),
            pltpu.VMEM((B * SQ_L, HL), jnp.bfloat16),
            pltpu.SemaphoreType.DMA((N_DEV // 2,)),
            pltpu.SemaphoreType.DMA((N_DEV // 2,)),
            pltpu.SemaphoreType.DMA((N_DEV // 2 - 1,)),
            pltpu.SemaphoreType.DMA((N_DEV // 2 - 1,)),
        ],
        compiler_params=pltpu.CompilerParams(collective_id=0),
    )(xb, wq, wot, bigkT, bigv)


# device time: 86044 ns/iter; 1.4380x vs baseline; 1.4380x over previous
import jax
import jax.numpy as jnp
from jax import lax
from jax.experimental import pallas as pl
from jax.experimental.pallas import tpu as pltpu

N_DEV = 16
HEADS_PER = 4
DH = 64
SQ_L = 128
SKV = 128
B = 2
E = 512
HL = HEADS_PER * DH


def kernel(x, Wq, K_ext, V_ext, Wo):
    xb = x.astype(jnp.bfloat16)
    wq = Wq.astype(jnp.bfloat16)
    wot = Wo.T.astype(jnp.bfloat16)
    kt = jnp.transpose(K_ext, (2, 0, 1, 3)).astype(jnp.bfloat16)
    vt = jnp.transpose(V_ext, (2, 0, 1, 3)).astype(jnp.bfloat16)

    def body(x_ref, wq_ref, wot_ref, kt_ref, vt_ref, out_ref,
             buf, acc, qs, cs, ws, send_sems, recv_sems, lsend_sems,
             lrecv_sems):
        my = lax.axis_index("i")
        left = lax.rem(my - 1 + N_DEV, N_DEV)
        right = lax.rem(my + 1, N_DEV)

        bsem = pltpu.get_barrier_semaphore()
        for nbr in (left, right):
            pl.semaphore_signal(bsem, inc=1, device_id=(nbr,),
                                device_id_type=pl.DeviceIdType.MESH)
        pl.semaphore_wait(bsem, 2)

        buf[my, 0] = wq_ref[...]
        buf[my, 1] = wot_ref[...]
        acc[...] = jnp.zeros_like(acc)

        x2d = x_ref[...].reshape(B * SQ_L, E)

        HK = HEADS_PER * SKV
        q_ids = my * SQ_L + lax.broadcasted_iota(jnp.int32, (SQ_L, HK), 0)
        k_ids = lax.rem(lax.broadcasted_iota(jnp.int32, (SQ_L, HK), 1), SKV)
        qb = q_ids // 64
        kb = k_ids // 64
        mask = (qb == kb) | (kb == 0) | (lax.rem(qb + kb, 3) == 0)
        neg_big = jnp.where(mask, 0.0, -1e9).astype(jnp.float32)

        mm = lambda a, b_, dims: lax.dot_general(
            a, b_, (dims, ((), ())), preferred_element_type=jnp.float32)

        neg = neg_big[:, :SKV]

        def compute_block(j):
            wq_j = buf[j, 0]
            qs[...] = mm(x2d, wq_j, ((1,), (0,))).astype(jnp.bfloat16)
            for b in range(B):
                for hh in range(HEADS_PER):
                    h_idx = j * HEADS_PER + hh
                    qbh = qs[b * SQ_L:(b + 1) * SQ_L, hh * DH:(hh + 1) * DH]
                    kbh = kt_ref[h_idx, b]
                    s = mm(qbh, kbh, ((1,), (1,))) * 0.125 + neg
                    e = jnp.exp(s)
                    w = (e / jnp.sum(e, axis=1, keepdims=True)).astype(
                        jnp.bfloat16)
                    vbh = vt_ref[h_idx, b]
                    c = mm(w, vbh, ((1,), (0,)))
                    cs[b * SQ_L:(b + 1) * SQ_L, hh * DH:(hh + 1) * DH] = (
                        c.astype(jnp.bfloat16))
            acc[...] = acc[...] + mm(cs[...], buf[j, 1], ((1,), (1,)))

        R_HOPS = N_DEV // 2
        L_HOPS = N_DEV - 1 - R_HOPS

        def hop(direction_dev, slot, s_sems, r_sems, h):
            rdma = pltpu.make_async_remote_copy(
                src_ref=buf.at[slot],
                dst_ref=buf.at[slot],
                send_sem=s_sems.at[h],
                recv_sem=r_sems.at[h],
                device_id=(direction_dev,),
                device_id_type=pl.DeviceIdType.MESH,
            )
            rdma.start()
            return rdma

        for h in range(R_HOPS):
            r_rdma = hop(right, lax.rem(my - h + N_DEV, N_DEV),
                         send_sems, recv_sems, h)
            l_rdma = None
            if h < L_HOPS:
                l_rdma = hop(left, lax.rem(my + h, N_DEV),
                             lsend_sems, lrecv_sems, h)
            if h == 0:
                compute_block(my)
            else:
                compute_block(lax.rem(my - h + N_DEV, N_DEV))
                compute_block(lax.rem(my + min(h, L_HOPS), N_DEV))
            r_rdma.wait_send()
            r_rdma.wait_recv()
            if l_rdma is not None:
                l_rdma.wait_send()
                l_rdma.wait_recv()
        compute_block(lax.rem(my + R_HOPS, N_DEV))

        out_ref[...] = acc[...].reshape(B, SQ_L, E)

    return pl.pallas_call(
        body,
        out_shape=jax.ShapeDtypeStruct((B, SQ_L, E), jnp.float32),
        in_specs=[pl.BlockSpec(memory_space=pltpu.VMEM)] * 5,
        out_specs=pl.BlockSpec(memory_space=pltpu.VMEM),
        scratch_shapes=[
            pltpu.VMEM((N_DEV, 2, E, HL), jnp.bfloat16),
            pltpu.VMEM((B * SQ_L, E), jnp.float32),
            pltpu.VMEM((B * SQ_L, HL), jnp.bfloat16),
            pltpu.VMEM((B * SQ_L, HL), jnp.bfloat16),
            pltpu.VMEM((B * SQ_L, HEADS_PER * SKV), jnp.bfloat16),
            pltpu.SemaphoreType.DMA((N_DEV // 2,)),
            pltpu.SemaphoreType.DMA((N_DEV // 2,)),
            pltpu.SemaphoreType.DMA((N_DEV // 2 - 1,)),
            pltpu.SemaphoreType.DMA((N_DEV // 2 - 1,)),
        ],
        compiler_params=pltpu.CompilerParams(collective_id=0),
    )(xb, wq, wot, kt, vt)


# device time: 58611 ns/iter; 2.1111x vs baseline; 1.4681x over previous
import jax
import jax.numpy as jnp
from jax import lax
from jax.experimental import pallas as pl
from jax.experimental.pallas import tpu as pltpu

N_DEV = 16
HEADS_PER = 4
DH = 64
SQ_L = 128
SKV = 128
B = 2
E = 512
HL = HEADS_PER * DH


def kernel(x, Wq, K_ext, V_ext, Wo):
    xb = x.astype(jnp.bfloat16)
    wq = Wq.astype(jnp.bfloat16)
    wot = Wo.T.astype(jnp.bfloat16)
    ktT2 = jnp.transpose(K_ext, (2, 3, 0, 1)).reshape(
        N_DEV * HEADS_PER, DH, B * SKV).astype(jnp.bfloat16)
    vt2 = jnp.transpose(V_ext, (2, 0, 1, 3)).reshape(
        N_DEV * HEADS_PER, B * SKV, DH).astype(jnp.bfloat16)

    def body(x_ref, wq_ref, wot_ref, ktT2_ref, vt2_ref, out_ref,
             buf, acc, qs, cs, ws, send_sems, recv_sems, lsend_sems,
             lrecv_sems):
        my = lax.axis_index("i")

        def ring_to_mesh(rr):
            rr = lax.rem(rr + 2 * N_DEV, N_DEV)
            q4 = rr // 4
            zz = lax.rem(rr, 4)
            return jnp.where(
                q4 == 0, 4 * zz,
                jnp.where(q4 == 1, 4 * (3 - zz) + 3,
                          jnp.where(q4 == 2, 4 * zz + 2, 4 * (3 - zz) + 1)))

        o = lax.rem(my, 4)
        z = my // 4
        r = jnp.where(o == 0, z,
                      jnp.where(o == 3, 7 - z,
                                jnp.where(o == 2, 8 + z, 15 - z)))
        left = ring_to_mesh(r - 1)
        right = ring_to_mesh(r + 1)

        bsem = pltpu.get_barrier_semaphore()
        for nbr in (left, right):
            pl.semaphore_signal(bsem, inc=1, device_id=(nbr,),
                                device_id_type=pl.DeviceIdType.MESH)
        pl.semaphore_wait(bsem, 2)

        buf[my, 0] = wq_ref[...]
        buf[my, 1] = wot_ref[...]
        acc[...] = jnp.zeros_like(acc)
        ws[...] = jnp.zeros_like(ws)

        x2d = x_ref[...].reshape(B * SQ_L, E)

        HK = HEADS_PER * SKV
        q_ids = my * SQ_L + lax.broadcasted_iota(jnp.int32, (SQ_L, HK), 0)
        k_ids = lax.rem(lax.broadcasted_iota(jnp.int32, (SQ_L, HK), 1), SKV)
        qb = q_ids // 64
        kb = k_ids // 64
        mask = (qb == kb) | (kb == 0) | (lax.rem(qb + kb, 3) == 0)
        neg_big = jnp.where(mask, 0.0, -1e9).astype(jnp.float32)

        mm = lambda a, b_, dims: lax.dot_general(
            a, b_, (dims, ((), ())), preferred_element_type=jnp.float32)

        neg = neg_big[:, :SKV]

        def compute_block(j):
            wq_j = buf[j, 0]
            qs[...] = mm(x2d, wq_j, ((1,), (0,))).astype(jnp.bfloat16)
            for hh in range(HEADS_PER):
                h_idx = j * HEADS_PER + hh
                s_all = mm(qs[:, hh * DH:(hh + 1) * DH], ktT2_ref[h_idx],
                           ((1,), (0,)))
                for b in range(B):
                    sl = slice(b * SQ_L, (b + 1) * SQ_L)
                    s = s_all[sl, sl] * 0.125 + neg
                    e = jnp.exp(s)
                    ws[sl, sl] = (e / jnp.sum(e, axis=1, keepdims=True)
                                  ).astype(jnp.bfloat16)
                c = mm(ws[...], vt2_ref[h_idx], ((1,), (0,)))
                cs[:, hh * DH:(hh + 1) * DH] = c.astype(jnp.bfloat16)
            acc[...] = acc[...] + mm(cs[...], buf[j, 1], ((1,), (1,)))

        R_HOPS = N_DEV // 2
        L_HOPS = N_DEV - 1 - R_HOPS

        def hop(direction_dev, slot, part, s_sems, r_sems, h):
            rdma = pltpu.make_async_remote_copy(
                src_ref=buf.at[slot, part],
                dst_ref=buf.at[slot, part],
                send_sem=s_sems.at[h, part],
                recv_sem=r_sems.at[h, part],
                device_id=(direction_dev,),
                device_id_type=pl.DeviceIdType.MESH,
            )
            rdma.start()
            return rdma

        def drain(rdma):
            rdma.wait_send()
            rdma.wait_recv()

        prev = {}
        for h in range(R_HOPS):
            rslot = ring_to_mesh(r - h)
            lslot = ring_to_mesh(r + h)
            ra = hop(right, rslot, 0, send_sems, recv_sems, h)
            la = hop(left, lslot, 0, lsend_sems, lrecv_sems, h) \
                if h < L_HOPS else None
            if h == 0:
                rb = hop(right, rslot, 1, send_sems, recv_sems, h)
                lb = hop(left, lslot, 1, lsend_sems, lrecv_sems, h)
                compute_block(my)
            else:
                drain(prev['rb'])
                rb = hop(right, rslot, 1, send_sems, recv_sems, h)
                compute_block(ring_to_mesh(r - h))
                lb = None
                if prev['lb'] is not None:
                    drain(prev['lb'])
                    if h < L_HOPS:
                        lb = hop(left, lslot, 1, lsend_sems, lrecv_sems, h)
                compute_block(ring_to_mesh(r + min(h, L_HOPS)))
            drain(ra)
            if la is not None:
                drain(la)
            prev = {'rb': rb, 'lb': lb}
        drain(prev['rb'])
        compute_block(ring_to_mesh(r + R_HOPS))

        out_ref[...] = acc[...].reshape(B, SQ_L, E)

    return pl.pallas_call(
        body,
        out_shape=jax.ShapeDtypeStruct((B, SQ_L, E), jnp.float32),
        in_specs=[pl.BlockSpec(memory_space=pltpu.VMEM)] * 5,
        out_specs=pl.BlockSpec(memory_space=pltpu.VMEM),
        scratch_shapes=[
            pltpu.VMEM((N_DEV, 2, E, HL), jnp.bfloat16),
            pltpu.VMEM((B * SQ_L, E), jnp.float32),
            pltpu.VMEM((B * SQ_L, HL), jnp.bfloat16),
            pltpu.VMEM((B * SQ_L, HL), jnp.bfloat16),
            pltpu.VMEM((B * SQ_L, B * SKV), jnp.bfloat16),
            pltpu.SemaphoreType.DMA((N_DEV // 2, 2)),
            pltpu.SemaphoreType.DMA((N_DEV // 2, 2)),
            pltpu.SemaphoreType.DMA((N_DEV // 2 - 1, 2)),
            pltpu.SemaphoreType.DMA((N_DEV // 2 - 1, 2)),
        ],
        compiler_params=pltpu.CompilerParams(collective_id=0),
    )(xb, wq, wot, ktT2, vt2)
